# baseline (speedup 1.0000x reference)
import functools

import jax
import jax.numpy as jnp
from jax import lax
from jax.experimental import pallas as pl
from jax.experimental.pallas import tpu as pltpu
from jax.experimental.pallas import tpu_sc as plsc

F32 = jnp.float32
BF16 = jnp.bfloat16


def _tables_body(nf_ref, wu_ref, wv_ref, tua_ref, tub_ref, tva_ref, tvb_ref):
    x = nf_ref[...].astype(BF16)
    yu = jnp.dot(x, wu_ref[0].astype(BF16), preferred_element_type=F32)
    yv = jnp.dot(x, wv_ref[0].astype(BF16), preferred_element_type=F32)
    h = yu.shape[1] // 2
    tua_ref[...] = yu[:, :h]
    tub_ref[...] = yu[:, h:]
    tva_ref[...] = yv[:, :h]
    tvb_ref[...] = yv[:, h:]


def _make_tables(nf, Wu, Wv, N, D, BN):
    nb = N // BN
    out = jax.ShapeDtypeStruct((2 * N, D // 2), F32)
    return pl.pallas_call(
        _tables_body,
        grid=(2, nb),
        in_specs=[
            pl.BlockSpec((BN, D), lambda t, i: (i, 0)),
            pl.BlockSpec((1, D, D), lambda t, i: (t, 0, 0)),
            pl.BlockSpec((1, D, D), lambda t, i: (t, 0, 0)),
        ],
        out_specs=[
            pl.BlockSpec((BN, D // 2), lambda t, i: (t * nb + i, 0)),
            pl.BlockSpec((BN, D // 2), lambda t, i: (t * nb + i, 0)),
            pl.BlockSpec((BN, D // 2), lambda t, i: (t * nb + i, 0)),
            pl.BlockSpec((BN, D // 2), lambda t, i: (t * nb + i, 0)),
        ],
        out_shape=[out, out, out, out],
    )(nf, Wu, Wv)


def _edge_body(ef_ref, wef_ref, b_ref, m8_ref, out_ref):
    x = ef_ref[...].astype(BF16)
    y0 = jnp.dot(x, wef_ref[0].astype(BF16), preferred_element_type=F32) + b_ref[0]
    y1 = jnp.dot(x, wef_ref[1].astype(BF16), preferred_element_type=F32) + b_ref[1]
    m = m8_ref[...].astype(F32)
    out_ref[...] = m * y0 + (1.0 - m) * y1


def _edge_mlp(ef, Wef, bns, m8, E, D, BE):
    return pl.pallas_call(
        _edge_body,
        grid=(E // BE,),
        in_specs=[
            pl.BlockSpec((BE, D), lambda i: (i, 0)),
            pl.BlockSpec((2, D, D), lambda i: (0, 0, 0)),
            pl.BlockSpec((2, 1, D), lambda i: (0, 0, 0)),
            pl.BlockSpec((BE, D), lambda i: (i, 0)),
        ],
        out_specs=pl.BlockSpec((BE, D), lambda i: (i, 0)),
        out_shape=jax.ShapeDtypeStruct((E, D), F32),
    )(ef, Wef, bns, m8)


def _sc_edge_body(N, E, D, C, ZR, R,
                  tua, tub, tva, tvb, iu_h, iv_h, efw_h, ef_h, zer_h,
                  uef_h, agga_h, aggb_h,
                  iu_t, iv_t, tu_v, tv_v, ew_v, ef_v, msg_v, uef_v, agg_sh,
                  sem_gi0, sem_gi1, sem_ge0, sem_ge1,
                  sem_om0, sem_om1, sem_os0, sem_os1, sem_ix):
    H = D // 2
    cid = lax.axis_index("c")
    sid = lax.axis_index("s")
    ept = E // 16
    nchunk = ept // C
    nz = (2 * N) // ZR
    nzq = (nz + 15) // 16
    sem_gi = (sem_gi0, sem_gi1)
    sem_ge = (sem_ge0, sem_ge1)
    sem_om = (sem_om0, sem_om1)
    sem_os = (sem_os0, sem_os1)

    def zc(q, _):
        qq = sid + q * 16

        @pl.when(qq < nz)
        def _():
            pltpu.sync_copy(zer_h, agg_sh.at[pl.ds(qq * ZR, ZR)])
        return 0
    lax.fori_loop(0, nzq, zc, 0)
    plsc.subcore_barrier()

    def pipeline(tab_u, tab_v, coff):
        row0 = sid * nchunk

        def iblock(b):
            return row0 + b * R, b % 3

        pltpu.sync_copy(iu_h.at[pl.ds(row0, R)], iu_t.at[0])
        pltpu.sync_copy(iv_h.at[pl.ds(row0, R)], iv_t.at[0])
        r1, s1 = iblock(1)
        pltpu.async_copy(iu_h.at[pl.ds(r1, R)], iu_t.at[s1], sem_ix)
        pltpu.async_copy(iv_h.at[pl.ds(r1, R)], iv_t.at[s1], sem_ix)

        def irow(k):
            return (k // R) % 3, k % R

        def gstart(k, p):
            base = sid * ept + k * C
            s, r = irow(k)
            pltpu.async_copy(tab_u.at[iu_t.at[s, r]], tu_v.at[p], sem_gi[p])
            pltpu.async_copy(tab_v.at[iv_t.at[s, r]], tv_v.at[p], sem_gi[p])
            pltpu.async_copy(efw_h.at[pl.ds(base, C), pl.ds(coff, H)],
                             ew_v.at[p], sem_ge[p])
            pltpu.async_copy(ef_h.at[pl.ds(base, C), pl.ds(coff, H)],
                             ef_v.at[p], sem_ge[p])

        def gdrain(k, p):
            base = sid * ept + k * C
            s, r = irow(k)
            pltpu.make_async_copy(tab_u.at[iu_t.at[s, r]], tu_v.at[p],
                                  sem_gi[p]).wait()
            pltpu.make_async_copy(tab_v.at[iv_t.at[s, r]], tv_v.at[p],
                                  sem_gi[p]).wait()
            pltpu.make_async_copy(efw_h.at[pl.ds(base, C), pl.ds(coff, H)],
                                  ew_v.at[p], sem_ge[p]).wait()
            pltpu.make_async_copy(ef_h.at[pl.ds(base, C), pl.ds(coff, H)],
                                  ef_v.at[p], sem_ge[p]).wait()

        def ostart(k, p):
            base = sid * ept + k * C
            s, r = irow(k)
            pltpu.async_copy(uef_v.at[p],
                             uef_h.at[pl.ds(base, C), pl.ds(coff, H)],
                             sem_om[p])
            pltpu.async_copy(msg_v.at[p], agg_sh.at[iv_t.at[s, r]],
                             sem_os[p], add=True)

        def odrain(k, p):
            base = sid * ept + k * C
            s, r = irow(k)
            pltpu.make_async_copy(uef_v.at[p],
                                  uef_h.at[pl.ds(base, C), pl.ds(coff, H)],
                                  sem_om[p]).wait()
            pltpu.make_async_copy(msg_v.at[p], agg_sh.at[iv_t.at[s, r]],
                                  sem_os[p]).wait()

        def compute(p):
            def row(i, _):
                for j in range(H // 16):
                    sl = pl.ds(j * 16, 16)
                    m = ew_v[p, i, sl] + tu_v[p, i, sl] + tv_v[p, i, sl]
                    msg_v[p, i, sl] = m
                    uef_v[p, i, sl] = m + ef_v[p, i, sl]
                return 0
            lax.fori_loop(0, C, row, 0)

        gstart(0, 0)

        def step(kk, _):
            k0 = kk * 2
            k1 = k0 + 1

            @pl.when(jnp.logical_and(lax.rem(k0, R) == R - 2,
                                     k0 + 2 < nchunk))
            def _():
                rb, sb = iblock((k0 + 2) // R)
                pltpu.make_async_copy(iu_h.at[pl.ds(rb, R)], iu_t.at[sb],
                                      sem_ix).wait()
                pltpu.make_async_copy(iv_h.at[pl.ds(rb, R)], iv_t.at[sb],
                                      sem_ix).wait()

            @pl.when(jnp.logical_and(lax.rem(k0, R) == 0,
                                     k0 + R < nchunk))
            def _():
                rb, sb = iblock(k0 // R + 1)
                pltpu.async_copy(iu_h.at[pl.ds(rb, R)], iu_t.at[sb], sem_ix)
                pltpu.async_copy(iv_h.at[pl.ds(rb, R)], iv_t.at[sb], sem_ix)

            @pl.when(kk >= 1)
            def _():
                odrain(k0 - 2, 0)
            gstart(k1, 1)
            gdrain(k0, 0)
            compute(0)
            ostart(k0, 0)

            @pl.when(kk >= 1)
            def _():
                odrain(k1 - 2, 1)

            @pl.when(kk < nchunk // 2 - 1)
            def _():
                gstart(k1 + 1, 0)
            gdrain(k1, 1)
            compute(1)
            ostart(k1, 1)
            return 0
        lax.fori_loop(0, nchunk // 2, step, 0)
        odrain(nchunk - 2, 0)
        odrain(nchunk - 1, 1)

    @pl.when(cid == 0)
    def _():
        pipeline(tua, tva, 0)

    @pl.when(cid == 1)
    def _():
        pipeline(tub, tvb, H)

    plsc.subcore_barrier()

    def co(q, _):
        qq = sid + q * 16

        @pl.when(qq < nz)
        def _():
            sl = pl.ds(qq * ZR, ZR)

            @pl.when(cid == 0)
            def _():
                pltpu.sync_copy(agg_sh.at[sl], agga_h.at[sl])

            @pl.when(cid == 1)
            def _():
                pltpu.sync_copy(agg_sh.at[sl], aggb_h.at[sl])
        return 0
    lax.fori_loop(0, nzq, co, 0)


def _sc_edges(tua, tub, tva, tvb, iu, iv, efw, ef, N, E, D):
    C = 40
    ZR = 200
    R = 10
    H = D // 2
    iu2 = iu.reshape(E // C, C)
    iv2 = iv.reshape(E // C, C)
    zer = jnp.zeros((ZR, H), F32)
    mesh = plsc.VectorSubcoreMesh(core_axis_name="c", subcore_axis_name="s",
                                  num_cores=2, num_subcores=16)
    eh = jax.ShapeDtypeStruct((E, D), F32)
    ah = jax.ShapeDtypeStruct((2 * N, H), F32)
    f = pl.kernel(
        functools.partial(_sc_edge_body, N, E, D, C, ZR, R),
        out_type=[eh, ah, ah],
        mesh=mesh,
        compiler_params=pltpu.CompilerParams(use_tc_tiling_on_sc=False),
        scratch_types=[
            pltpu.VMEM((3, R, C), jnp.int32),
            pltpu.VMEM((3, R, C), jnp.int32),
            pltpu.VMEM((2, C, H), F32),
            pltpu.VMEM((2, C, H), F32),
            pltpu.VMEM((2, C, H), F32),
            pltpu.VMEM((2, C, H), F32),
            pltpu.VMEM((2, C, H), F32),
            pltpu.VMEM((2, C, H), F32),
            pltpu.VMEM_SHARED((2 * N, H), F32),
            pltpu.SemaphoreType.DMA,
            pltpu.SemaphoreType.DMA,
            pltpu.SemaphoreType.DMA,
            pltpu.SemaphoreType.DMA,
            pltpu.SemaphoreType.DMA,
            pltpu.SemaphoreType.DMA,
            pltpu.SemaphoreType.DMA,
            pltpu.SemaphoreType.DMA,
            pltpu.SemaphoreType.DMA,
        ],
    )
    return f(tua, tub, tva, tvb, iu2, iv2, efw, ef, zer)


def _uef_body(ef_ref, m_ref, out_ref):
    out_ref[...] = m_ref[...] + ef_ref[...]


def _uef_add(ef, msg, E, D, BE):
    return pl.pallas_call(
        _uef_body,
        grid=(E // BE,),
        in_specs=[
            pl.BlockSpec((BE, D), lambda i: (i, 0)),
            pl.BlockSpec((BE, D), lambda i: (i, 0)),
        ],
        out_specs=pl.BlockSpec((BE, D), lambda i: (i, 0)),
        out_shape=jax.ShapeDtypeStruct((E, D), F32),
    )(ef, msg)


def _node_body(nf_ref, a0a_ref, a0b_ref, a1a_ref, a1b_ref,
               wc_ref, wd_ref, bc_ref, bd_ref, md_ref, out_ref):
    x = jnp.concatenate([nf_ref[...], a0a_ref[...], a0b_ref[...],
                         a1a_ref[...], a1b_ref[...]], axis=1).astype(BF16)
    yc = jnp.dot(x, wc_ref[...].astype(BF16), preferred_element_type=F32) + bc_ref[...]
    yd = jnp.dot(x, wd_ref[...].astype(BF16), preferred_element_type=F32) + bd_ref[...]
    m = md_ref[...]
    out_ref[...] = m * yc + (1.0 - m) * yd + nf_ref[...]


def _node_mlp(nf, agga, aggb, Wc, Wd, bc, bd, md, N, D, BN):
    nb = N // BN
    h = D // 2
    return pl.pallas_call(
        _node_body,
        grid=(nb,),
        in_specs=[
            pl.BlockSpec((BN, D), lambda i: (i, 0)),
            pl.BlockSpec((BN, h), lambda i: (i, 0)),
            pl.BlockSpec((BN, h), lambda i: (i, 0)),
            pl.BlockSpec((BN, h), lambda i: (i + nb, 0)),
            pl.BlockSpec((BN, h), lambda i: (i + nb, 0)),
            pl.BlockSpec((3 * D, D), lambda i: (0, 0)),
            pl.BlockSpec((3 * D, D), lambda i: (0, 0)),
            pl.BlockSpec((1, D), lambda i: (0, 0)),
            pl.BlockSpec((1, D), lambda i: (0, 0)),
            pl.BlockSpec((BN, 1), lambda i: (i, 0)),
        ],
        out_specs=pl.BlockSpec((BN, D), lambda i: (i, 0)),
        out_shape=jax.ShapeDtypeStruct((N, D), F32),
    )(nf, agga, aggb, agga, aggb, Wc, Wd, bc, bd, md)


def kernel(nf, ef, edge_index, etypes, is_dummy,
           W_outer, b_outer, W_inter, b_inter,
           W_cust, b_cust, W_dummy, b_dummy):
    N, D = nf.shape
    E = ef.shape[0]

    src = edge_index[0]
    dst = edge_index[1]
    iu = etypes * N + src
    iv = etypes * N + dst
    m8 = jnp.broadcast_to((etypes == 0).astype(BF16).reshape(E, 1), (E, D))
    md = (is_dummy == 0).astype(F32).reshape(N, 1)

    Wef = jnp.stack([W_outer[:D], W_inter[:D]])
    Wu = jnp.stack([W_outer[D:2 * D], W_inter[D:2 * D]])
    Wv = jnp.stack([W_outer[2 * D:], W_inter[2 * D:]])
    bns = jnp.stack([b_outer, b_inter]).reshape(2, 1, D)

    tua, tub, tva, tvb = _make_tables(nf, Wu, Wv, N, D, BN=N // 10)
    efw = _edge_mlp(ef, Wef, bns, m8, E, D, BE=3200)
    uef, agga, aggb = _sc_edges(tua, tub, tva, tvb, iu, iv, efw, ef, N, E, D)
    unf = _node_mlp(nf, agga, aggb, W_cust, W_dummy,
                    b_cust.reshape(1, D), b_dummy.reshape(1, D),
                    md, N, D, BN=N // 5)
    return (unf, uef)


# speedup vs baseline: 1.0195x; 1.0195x over previous
import functools

import jax
import jax.numpy as jnp
from jax import lax
from jax.experimental import pallas as pl
from jax.experimental.pallas import tpu as pltpu
from jax.experimental.pallas import tpu_sc as plsc

F32 = jnp.float32
BF16 = jnp.bfloat16


def _tables_body(nf_ref, wu_ref, wv_ref, tua_ref, tub_ref, tva_ref, tvb_ref):
    x = nf_ref[...].astype(BF16)
    yu = jnp.dot(x, wu_ref[0].astype(BF16), preferred_element_type=F32)
    yv = jnp.dot(x, wv_ref[0].astype(BF16), preferred_element_type=F32)
    h = yu.shape[1] // 2
    tua_ref[...] = yu[:, :h]
    tub_ref[...] = yu[:, h:]
    tva_ref[...] = yv[:, :h]
    tvb_ref[...] = yv[:, h:]


def _make_tables(nf, Wu, Wv, N, D, BN):
    nb = N // BN
    out = jax.ShapeDtypeStruct((2 * N, D // 2), F32)
    return pl.pallas_call(
        _tables_body,
        grid=(2, nb),
        in_specs=[
            pl.BlockSpec((BN, D), lambda t, i: (i, 0)),
            pl.BlockSpec((1, D, D), lambda t, i: (t, 0, 0)),
            pl.BlockSpec((1, D, D), lambda t, i: (t, 0, 0)),
        ],
        out_specs=[
            pl.BlockSpec((BN, D // 2), lambda t, i: (t * nb + i, 0)),
            pl.BlockSpec((BN, D // 2), lambda t, i: (t * nb + i, 0)),
            pl.BlockSpec((BN, D // 2), lambda t, i: (t * nb + i, 0)),
            pl.BlockSpec((BN, D // 2), lambda t, i: (t * nb + i, 0)),
        ],
        out_shape=[out, out, out, out],
    )(nf, Wu, Wv)


def _edge_body(ef_ref, wef_ref, b_ref, m8_ref, out_ref):
    x = ef_ref[...].astype(BF16)
    y0 = jnp.dot(x, wef_ref[0].astype(BF16), preferred_element_type=F32) + b_ref[0]
    y1 = jnp.dot(x, wef_ref[1].astype(BF16), preferred_element_type=F32) + b_ref[1]
    m = m8_ref[...]
    out_ref[...] = m * y0 + (1.0 - m) * y1


def _edge_mlp(ef, Wef, bns, m8, E, D, BE):
    return pl.pallas_call(
        _edge_body,
        grid=(E // BE,),
        in_specs=[
            pl.BlockSpec((BE, D), lambda i: (i, 0)),
            pl.BlockSpec((2, D, D), lambda i: (0, 0, 0)),
            pl.BlockSpec((2, 1, D), lambda i: (0, 0, 0)),
            pl.BlockSpec((BE, D), lambda i: (i, 0)),
        ],
        out_specs=pl.BlockSpec((BE, D), lambda i: (i, 0)),
        out_shape=jax.ShapeDtypeStruct((E, D), F32),
    )(ef, Wef, bns, m8)


def _sc_edge_body(N, E, D, C, ZR, R,
                  tua, tub, tva, tvb, iu_h, iv_h, efw_h, ef_h, zer_h,
                  uef_h, agga_h, aggb_h,
                  iu_t, iv_t, tu_v, tv_v, ew_v, ef_v, msg_v, uef_v, agg_sh,
                  sem_gi0, sem_gi1, sem_ge0, sem_ge1,
                  sem_om0, sem_om1, sem_os0, sem_os1, sem_ix):
    H = D // 2
    cid = lax.axis_index("c")
    sid = lax.axis_index("s")
    ept = E // 16
    nchunk = ept // C
    nz = (2 * N) // ZR
    nzq = (nz + 15) // 16
    sem_gi = (sem_gi0, sem_gi1)
    sem_ge = (sem_ge0, sem_ge1)
    sem_om = (sem_om0, sem_om1)
    sem_os = (sem_os0, sem_os1)

    def zc(q, _):
        qq = sid + q * 16

        @pl.when(qq < nz)
        def _():
            pltpu.sync_copy(zer_h, agg_sh.at[pl.ds(qq * ZR, ZR)])
        return 0
    lax.fori_loop(0, nzq, zc, 0)
    plsc.subcore_barrier()

    def pipeline(tab_u, tab_v, coff):
        row0 = sid * nchunk

        def iblock(b):
            return row0 + b * R, b % 3

        pltpu.sync_copy(iu_h.at[pl.ds(row0, R)], iu_t.at[0])
        pltpu.sync_copy(iv_h.at[pl.ds(row0, R)], iv_t.at[0])
        r1, s1 = iblock(1)
        pltpu.async_copy(iu_h.at[pl.ds(r1, R)], iu_t.at[s1], sem_ix)
        pltpu.async_copy(iv_h.at[pl.ds(r1, R)], iv_t.at[s1], sem_ix)

        def irow(k):
            return (k // R) % 3, k % R

        def gstart(k, p):
            base = sid * ept + k * C
            s, r = irow(k)
            pltpu.async_copy(tab_u.at[iu_t.at[s, r]], tu_v.at[p], sem_gi[p])
            pltpu.async_copy(tab_v.at[iv_t.at[s, r]], tv_v.at[p], sem_gi[p])
            pltpu.async_copy(efw_h.at[pl.ds(base, C), pl.ds(coff, H)],
                             ew_v.at[p], sem_ge[p])
            pltpu.async_copy(ef_h.at[pl.ds(base, C), pl.ds(coff, H)],
                             ef_v.at[p], sem_ge[p])

        def gdrain(k, p):
            base = sid * ept + k * C
            s, r = irow(k)
            pltpu.make_async_copy(tab_u.at[iu_t.at[s, r]], tu_v.at[p],
                                  sem_gi[p]).wait()
            pltpu.make_async_copy(tab_v.at[iv_t.at[s, r]], tv_v.at[p],
                                  sem_gi[p]).wait()
            pltpu.make_async_copy(efw_h.at[pl.ds(base, C), pl.ds(coff, H)],
                                  ew_v.at[p], sem_ge[p]).wait()
            pltpu.make_async_copy(ef_h.at[pl.ds(base, C), pl.ds(coff, H)],
                                  ef_v.at[p], sem_ge[p]).wait()

        def ostart(k, p):
            base = sid * ept + k * C
            s, r = irow(k)
            pltpu.async_copy(uef_v.at[p],
                             uef_h.at[pl.ds(base, C), pl.ds(coff, H)],
                             sem_om[p])
            pltpu.async_copy(msg_v.at[p], agg_sh.at[iv_t.at[s, r]],
                             sem_os[p], add=True)

        def odrain(k, p):
            base = sid * ept + k * C
            s, r = irow(k)
            pltpu.make_async_copy(uef_v.at[p],
                                  uef_h.at[pl.ds(base, C), pl.ds(coff, H)],
                                  sem_om[p]).wait()
            pltpu.make_async_copy(msg_v.at[p], agg_sh.at[iv_t.at[s, r]],
                                  sem_os[p]).wait()

        def compute(p):
            def row(i, _):
                for j in range(H // 16):
                    sl = pl.ds(j * 16, 16)
                    m = ew_v[p, i, sl] + tu_v[p, i, sl] + tv_v[p, i, sl]
                    msg_v[p, i, sl] = m
                    uef_v[p, i, sl] = m + ef_v[p, i, sl]
                return 0
            lax.fori_loop(0, C, row, 0)

        gstart(0, 0)

        def step(kk, _):
            k0 = kk * 2
            k1 = k0 + 1

            @pl.when(jnp.logical_and(lax.rem(k0, R) == R - 2,
                                     k0 + 2 < nchunk))
            def _():
                rb, sb = iblock((k0 + 2) // R)
                pltpu.make_async_copy(iu_h.at[pl.ds(rb, R)], iu_t.at[sb],
                                      sem_ix).wait()
                pltpu.make_async_copy(iv_h.at[pl.ds(rb, R)], iv_t.at[sb],
                                      sem_ix).wait()

            @pl.when(jnp.logical_and(lax.rem(k0, R) == 0,
                                     k0 + R < nchunk))
            def _():
                rb, sb = iblock(k0 // R + 1)
                pltpu.async_copy(iu_h.at[pl.ds(rb, R)], iu_t.at[sb], sem_ix)
                pltpu.async_copy(iv_h.at[pl.ds(rb, R)], iv_t.at[sb], sem_ix)

            @pl.when(kk >= 1)
            def _():
                odrain(k0 - 2, 0)
            gstart(k1, 1)
            gdrain(k0, 0)
            compute(0)
            ostart(k0, 0)

            @pl.when(kk >= 1)
            def _():
                odrain(k1 - 2, 1)

            @pl.when(kk < nchunk // 2 - 1)
            def _():
                gstart(k1 + 1, 0)
            gdrain(k1, 1)
            compute(1)
            ostart(k1, 1)
            return 0
        lax.fori_loop(0, nchunk // 2, step, 0)
        odrain(nchunk - 2, 0)
        odrain(nchunk - 1, 1)

    @pl.when(cid == 0)
    def _():
        pipeline(tua, tva, 0)

    @pl.when(cid == 1)
    def _():
        pipeline(tub, tvb, H)

    plsc.subcore_barrier()

    def co(q, _):
        qq = sid + q * 16

        @pl.when(qq < nz)
        def _():
            sl = pl.ds(qq * ZR, ZR)

            @pl.when(cid == 0)
            def _():
                pltpu.sync_copy(agg_sh.at[sl], agga_h.at[sl])

            @pl.when(cid == 1)
            def _():
                pltpu.sync_copy(agg_sh.at[sl], aggb_h.at[sl])
        return 0
    lax.fori_loop(0, nzq, co, 0)


def _sc_edges(tua, tub, tva, tvb, iu, iv, efw, ef, N, E, D):
    C = 50
    ZR = 200
    R = 10
    H = D // 2
    iu2 = iu.reshape(E // C, C)
    iv2 = iv.reshape(E // C, C)
    zer = jnp.zeros((ZR, H), F32)
    mesh = plsc.VectorSubcoreMesh(core_axis_name="c", subcore_axis_name="s",
                                  num_cores=2, num_subcores=16)
    eh = jax.ShapeDtypeStruct((E, D), F32)
    ah = jax.ShapeDtypeStruct((2 * N, H), F32)
    f = pl.kernel(
        functools.partial(_sc_edge_body, N, E, D, C, ZR, R),
        out_type=[eh, ah, ah],
        mesh=mesh,
        compiler_params=pltpu.CompilerParams(use_tc_tiling_on_sc=False),
        scratch_types=[
            pltpu.VMEM((3, R, C), jnp.int32),
            pltpu.VMEM((3, R, C), jnp.int32),
            pltpu.VMEM((2, C, H), F32),
            pltpu.VMEM((2, C, H), F32),
            pltpu.VMEM((2, C, H), F32),
            pltpu.VMEM((2, C, H), F32),
            pltpu.VMEM((2, C, H), F32),
            pltpu.VMEM((2, C, H), F32),
            pltpu.VMEM_SHARED((2 * N, H), F32),
            pltpu.SemaphoreType.DMA,
            pltpu.SemaphoreType.DMA,
            pltpu.SemaphoreType.DMA,
            pltpu.SemaphoreType.DMA,
            pltpu.SemaphoreType.DMA,
            pltpu.SemaphoreType.DMA,
            pltpu.SemaphoreType.DMA,
            pltpu.SemaphoreType.DMA,
            pltpu.SemaphoreType.DMA,
        ],
    )
    return f(tua, tub, tva, tvb, iu2, iv2, efw, ef, zer)


def _uef_body(ef_ref, m_ref, out_ref):
    out_ref[...] = m_ref[...] + ef_ref[...]


def _uef_add(ef, msg, E, D, BE):
    return pl.pallas_call(
        _uef_body,
        grid=(E // BE,),
        in_specs=[
            pl.BlockSpec((BE, D), lambda i: (i, 0)),
            pl.BlockSpec((BE, D), lambda i: (i, 0)),
        ],
        out_specs=pl.BlockSpec((BE, D), lambda i: (i, 0)),
        out_shape=jax.ShapeDtypeStruct((E, D), F32),
    )(ef, msg)


def _node_body(nf_ref, a0a_ref, a0b_ref, a1a_ref, a1b_ref,
               wc_ref, wd_ref, bc_ref, bd_ref, md_ref, out_ref):
    x = jnp.concatenate([nf_ref[...], a0a_ref[...], a0b_ref[...],
                         a1a_ref[...], a1b_ref[...]], axis=1).astype(BF16)
    yc = jnp.dot(x, wc_ref[...].astype(BF16), preferred_element_type=F32) + bc_ref[...]
    yd = jnp.dot(x, wd_ref[...].astype(BF16), preferred_element_type=F32) + bd_ref[...]
    m = md_ref[...]
    out_ref[...] = m * yc + (1.0 - m) * yd + nf_ref[...]


def _node_mlp(nf, agga, aggb, Wc, Wd, bc, bd, md, N, D, BN):
    nb = N // BN
    h = D // 2
    return pl.pallas_call(
        _node_body,
        grid=(nb,),
        in_specs=[
            pl.BlockSpec((BN, D), lambda i: (i, 0)),
            pl.BlockSpec((BN, h), lambda i: (i, 0)),
            pl.BlockSpec((BN, h), lambda i: (i, 0)),
            pl.BlockSpec((BN, h), lambda i: (i + nb, 0)),
            pl.BlockSpec((BN, h), lambda i: (i + nb, 0)),
            pl.BlockSpec((3 * D, D), lambda i: (0, 0)),
            pl.BlockSpec((3 * D, D), lambda i: (0, 0)),
            pl.BlockSpec((1, D), lambda i: (0, 0)),
            pl.BlockSpec((1, D), lambda i: (0, 0)),
            pl.BlockSpec((BN, 1), lambda i: (i, 0)),
        ],
        out_specs=pl.BlockSpec((BN, D), lambda i: (i, 0)),
        out_shape=jax.ShapeDtypeStruct((N, D), F32),
    )(nf, agga, aggb, agga, aggb, Wc, Wd, bc, bd, md)


def kernel(nf, ef, edge_index, etypes, is_dummy,
           W_outer, b_outer, W_inter, b_inter,
           W_cust, b_cust, W_dummy, b_dummy):
    N, D = nf.shape
    E = ef.shape[0]

    src = edge_index[0]
    dst = edge_index[1]
    iu = etypes * N + src
    iv = etypes * N + dst
    m8 = jnp.broadcast_to((etypes == 0).astype(F32).reshape(E, 1), (E, D))
    md = (is_dummy == 0).astype(F32).reshape(N, 1)

    Wef = jnp.stack([W_outer[:D], W_inter[:D]])
    Wu = jnp.stack([W_outer[D:2 * D], W_inter[D:2 * D]])
    Wv = jnp.stack([W_outer[2 * D:], W_inter[2 * D:]])
    bns = jnp.stack([b_outer, b_inter]).reshape(2, 1, D)

    tua, tub, tva, tvb = _make_tables(nf, Wu, Wv, N, D, BN=N // 10)
    efw = _edge_mlp(ef, Wef, bns, m8, E, D, BE=3200)
    uef, agga, aggb = _sc_edges(tua, tub, tva, tvb, iu, iv, efw, ef, N, E, D)
    unf = _node_mlp(nf, agga, aggb, W_cust, W_dummy,
                    b_cust.reshape(1, D), b_dummy.reshape(1, D),
                    md, N, D, BN=N // 5)
    return (unf, uef)


# speedup vs baseline: 1.0784x; 1.0578x over previous
import functools

import jax
import jax.numpy as jnp
from jax import lax
from jax.experimental import pallas as pl
from jax.experimental.pallas import tpu as pltpu
from jax.experimental.pallas import tpu_sc as plsc

F32 = jnp.float32
BF16 = jnp.bfloat16


def _tables_body(nf_ref, wu_ref, wv_ref, tua_ref, tub_ref, tva_ref, tvb_ref):
    x = nf_ref[...].astype(BF16)
    yu = jnp.dot(x, wu_ref[0].astype(BF16), preferred_element_type=F32)
    yv = jnp.dot(x, wv_ref[0].astype(BF16), preferred_element_type=F32)
    h = yu.shape[1] // 2
    tua_ref[...] = yu[:, :h]
    tub_ref[...] = yu[:, h:]
    tva_ref[...] = yv[:, :h]
    tvb_ref[...] = yv[:, h:]


def _make_tables(nf, Wu, Wv, N, D, BN):
    nb = N // BN
    out = jax.ShapeDtypeStruct((2 * N, D // 2), F32)
    return pl.pallas_call(
        _tables_body,
        grid=(2, nb),
        in_specs=[
            pl.BlockSpec((BN, D), lambda t, i: (i, 0)),
            pl.BlockSpec((1, D, D), lambda t, i: (t, 0, 0)),
            pl.BlockSpec((1, D, D), lambda t, i: (t, 0, 0)),
        ],
        out_specs=[
            pl.BlockSpec((BN, D // 2), lambda t, i: (t * nb + i, 0)),
            pl.BlockSpec((BN, D // 2), lambda t, i: (t * nb + i, 0)),
            pl.BlockSpec((BN, D // 2), lambda t, i: (t * nb + i, 0)),
            pl.BlockSpec((BN, D // 2), lambda t, i: (t * nb + i, 0)),
        ],
        out_shape=[out, out, out, out],
    )(nf, Wu, Wv)


def _edge_body(ef_ref, wef_ref, b_ref, out0_ref, out1_ref):
    x = ef_ref[...].astype(BF16)
    out0_ref[...] = jnp.dot(x, wef_ref[0].astype(BF16),
                            preferred_element_type=F32) + b_ref[0]
    out1_ref[...] = jnp.dot(x, wef_ref[1].astype(BF16),
                            preferred_element_type=F32) + b_ref[1]


def _edge_mlp(ef, Wef, bns, E, D, BE):
    out = jax.ShapeDtypeStruct((E, D), F32)
    return pl.pallas_call(
        _edge_body,
        grid=(E // BE,),
        in_specs=[
            pl.BlockSpec((BE, D), lambda i: (i, 0)),
            pl.BlockSpec((2, D, D), lambda i: (0, 0, 0)),
            pl.BlockSpec((2, 1, D), lambda i: (0, 0, 0)),
        ],
        out_specs=[
            pl.BlockSpec((BE, D), lambda i: (i, 0)),
            pl.BlockSpec((BE, D), lambda i: (i, 0)),
        ],
        out_shape=[out, out],
    )(ef, Wef, bns)


def _sc_edge_body(N, E, D, C, ZR, R,
                  tua, tub, tva, tvb, iu_h, iv_h, efw0_h, efw1_h, mt_h,
                  ef_h, zer_h,
                  uef_h, agga_h, aggb_h,
                  iu_t, iv_t, tu_v, tv_v, ew_v, ew1_v, mk_v, ef_v,
                  msg_v, uef_v, agg_sh,
                  sem_gi0, sem_gi1, sem_ge0, sem_ge1,
                  sem_om0, sem_om1, sem_os0, sem_os1, sem_ix):
    H = D // 2
    cid = lax.axis_index("c")
    sid = lax.axis_index("s")
    ept = E // 16
    nchunk = ept // C
    nz = (2 * N) // ZR
    nzq = (nz + 15) // 16
    sem_gi = (sem_gi0, sem_gi1)
    sem_ge = (sem_ge0, sem_ge1)
    sem_om = (sem_om0, sem_om1)
    sem_os = (sem_os0, sem_os1)

    def zc(q, _):
        qq = sid + q * 16

        @pl.when(qq < nz)
        def _():
            pltpu.sync_copy(zer_h, agg_sh.at[pl.ds(qq * ZR, ZR)])
        return 0
    lax.fori_loop(0, nzq, zc, 0)
    plsc.subcore_barrier()

    def pipeline(tab_u, tab_v, coff):
        row0 = sid * nchunk

        def iblock(b):
            return row0 + b * R, b % 3

        pltpu.sync_copy(iu_h.at[pl.ds(row0, R)], iu_t.at[0])
        pltpu.sync_copy(iv_h.at[pl.ds(row0, R)], iv_t.at[0])
        r1, s1 = iblock(1)
        pltpu.async_copy(iu_h.at[pl.ds(r1, R)], iu_t.at[s1], sem_ix)
        pltpu.async_copy(iv_h.at[pl.ds(r1, R)], iv_t.at[s1], sem_ix)

        def irow(k):
            return (k // R) % 3, k % R

        def gstart(k, p):
            base = sid * ept + k * C
            s, r = irow(k)
            pltpu.async_copy(tab_u.at[iu_t.at[s, r]], tu_v.at[p], sem_gi[p])
            pltpu.async_copy(tab_v.at[iv_t.at[s, r]], tv_v.at[p], sem_gi[p])
            pltpu.async_copy(mt_h.at[iu_t.at[s, r]], mk_v.at[p], sem_gi[p])
            pltpu.async_copy(efw0_h.at[pl.ds(base, C), pl.ds(coff, H)],
                             ew_v.at[p], sem_ge[p])
            pltpu.async_copy(efw1_h.at[pl.ds(base, C), pl.ds(coff, H)],
                             ew1_v.at[p], sem_ge[p])
            pltpu.async_copy(ef_h.at[pl.ds(base, C), pl.ds(coff, H)],
                             ef_v.at[p], sem_ge[p])

        def gdrain(k, p):
            base = sid * ept + k * C
            s, r = irow(k)
            pltpu.make_async_copy(tab_u.at[iu_t.at[s, r]], tu_v.at[p],
                                  sem_gi[p]).wait()
            pltpu.make_async_copy(tab_v.at[iv_t.at[s, r]], tv_v.at[p],
                                  sem_gi[p]).wait()
            pltpu.make_async_copy(mt_h.at[iu_t.at[s, r]], mk_v.at[p],
                                  sem_gi[p]).wait()
            pltpu.make_async_copy(efw0_h.at[pl.ds(base, C), pl.ds(coff, H)],
                                  ew_v.at[p], sem_ge[p]).wait()
            pltpu.make_async_copy(efw1_h.at[pl.ds(base, C), pl.ds(coff, H)],
                                  ew1_v.at[p], sem_ge[p]).wait()
            pltpu.make_async_copy(ef_h.at[pl.ds(base, C), pl.ds(coff, H)],
                                  ef_v.at[p], sem_ge[p]).wait()

        def ostart(k, p):
            base = sid * ept + k * C
            s, r = irow(k)
            pltpu.async_copy(uef_v.at[p],
                             uef_h.at[pl.ds(base, C), pl.ds(coff, H)],
                             sem_om[p])
            pltpu.async_copy(msg_v.at[p], agg_sh.at[iv_t.at[s, r]],
                             sem_os[p], add=True)

        def odrain(k, p):
            base = sid * ept + k * C
            s, r = irow(k)
            pltpu.make_async_copy(uef_v.at[p],
                                  uef_h.at[pl.ds(base, C), pl.ds(coff, H)],
                                  sem_om[p]).wait()
            pltpu.make_async_copy(msg_v.at[p], agg_sh.at[iv_t.at[s, r]],
                                  sem_os[p]).wait()

        def compute(p):
            def row(i, _):
                mb = mk_v[p, i, :] > 0.5
                for j in range(H // 16):
                    sl = pl.ds(j * 16, 16)
                    ew = jnp.where(mb, ew_v[p, i, sl], ew1_v[p, i, sl])
                    m = ew + tu_v[p, i, sl] + tv_v[p, i, sl]
                    msg_v[p, i, sl] = m
                    uef_v[p, i, sl] = m + ef_v[p, i, sl]
                return 0
            lax.fori_loop(0, C, row, 0)

        gstart(0, 0)

        def step(kk, _):
            k0 = kk * 2
            k1 = k0 + 1

            @pl.when(jnp.logical_and(lax.rem(k0, R) == R - 2,
                                     k0 + 2 < nchunk))
            def _():
                rb, sb = iblock((k0 + 2) // R)
                pltpu.make_async_copy(iu_h.at[pl.ds(rb, R)], iu_t.at[sb],
                                      sem_ix).wait()
                pltpu.make_async_copy(iv_h.at[pl.ds(rb, R)], iv_t.at[sb],
                                      sem_ix).wait()

            @pl.when(jnp.logical_and(lax.rem(k0, R) == 0,
                                     k0 + R < nchunk))
            def _():
                rb, sb = iblock(k0 // R + 1)
                pltpu.async_copy(iu_h.at[pl.ds(rb, R)], iu_t.at[sb], sem_ix)
                pltpu.async_copy(iv_h.at[pl.ds(rb, R)], iv_t.at[sb], sem_ix)

            @pl.when(kk >= 1)
            def _():
                odrain(k0 - 2, 0)
            gstart(k1, 1)
            gdrain(k0, 0)
            compute(0)
            ostart(k0, 0)

            @pl.when(kk >= 1)
            def _():
                odrain(k1 - 2, 1)

            @pl.when(kk < nchunk // 2 - 1)
            def _():
                gstart(k1 + 1, 0)
            gdrain(k1, 1)
            compute(1)
            ostart(k1, 1)
            return 0
        lax.fori_loop(0, nchunk // 2, step, 0)
        odrain(nchunk - 2, 0)
        odrain(nchunk - 1, 1)

    @pl.when(cid == 0)
    def _():
        pipeline(tua, tva, 0)

    @pl.when(cid == 1)
    def _():
        pipeline(tub, tvb, H)

    plsc.subcore_barrier()

    def co(q, _):
        qq = sid + q * 16

        @pl.when(qq < nz)
        def _():
            sl = pl.ds(qq * ZR, ZR)

            @pl.when(cid == 0)
            def _():
                pltpu.sync_copy(agg_sh.at[sl], agga_h.at[sl])

            @pl.when(cid == 1)
            def _():
                pltpu.sync_copy(agg_sh.at[sl], aggb_h.at[sl])
        return 0
    lax.fori_loop(0, nzq, co, 0)


def _sc_edges(tua, tub, tva, tvb, iu, iv, efw0, efw1, mt, ef, N, E, D):
    C = 40
    ZR = 200
    R = 10
    H = D // 2
    iu2 = iu.reshape(E // C, C)
    iv2 = iv.reshape(E // C, C)
    zer = jnp.zeros((ZR, H), F32)
    mesh = plsc.VectorSubcoreMesh(core_axis_name="c", subcore_axis_name="s",
                                  num_cores=2, num_subcores=16)
    eh = jax.ShapeDtypeStruct((E, D), F32)
    ah = jax.ShapeDtypeStruct((2 * N, H), F32)
    f = pl.kernel(
        functools.partial(_sc_edge_body, N, E, D, C, ZR, R),
        out_type=[eh, ah, ah],
        mesh=mesh,
        compiler_params=pltpu.CompilerParams(use_tc_tiling_on_sc=False),
        scratch_types=[
            pltpu.VMEM((3, R, C), jnp.int32),
            pltpu.VMEM((3, R, C), jnp.int32),
            pltpu.VMEM((2, C, H), F32),
            pltpu.VMEM((2, C, H), F32),
            pltpu.VMEM((2, C, H), F32),
            pltpu.VMEM((2, C, H), F32),
            pltpu.VMEM((2, C, 16), F32),
            pltpu.VMEM((2, C, H), F32),
            pltpu.VMEM((2, C, H), F32),
            pltpu.VMEM((2, C, H), F32),
            pltpu.VMEM_SHARED((2 * N, H), F32),
            pltpu.SemaphoreType.DMA,
            pltpu.SemaphoreType.DMA,
            pltpu.SemaphoreType.DMA,
            pltpu.SemaphoreType.DMA,
            pltpu.SemaphoreType.DMA,
            pltpu.SemaphoreType.DMA,
            pltpu.SemaphoreType.DMA,
            pltpu.SemaphoreType.DMA,
            pltpu.SemaphoreType.DMA,
        ],
    )
    return f(tua, tub, tva, tvb, iu2, iv2, efw0, efw1, mt, ef, zer)


def _uef_body(ef_ref, m_ref, out_ref):
    out_ref[...] = m_ref[...] + ef_ref[...]


def _uef_add(ef, msg, E, D, BE):
    return pl.pallas_call(
        _uef_body,
        grid=(E // BE,),
        in_specs=[
            pl.BlockSpec((BE, D), lambda i: (i, 0)),
            pl.BlockSpec((BE, D), lambda i: (i, 0)),
        ],
        out_specs=pl.BlockSpec((BE, D), lambda i: (i, 0)),
        out_shape=jax.ShapeDtypeStruct((E, D), F32),
    )(ef, msg)


def _node_body(nf_ref, a0a_ref, a0b_ref, a1a_ref, a1b_ref,
               wc_ref, wd_ref, bc_ref, bd_ref, md_ref, out_ref):
    x = jnp.concatenate([nf_ref[...], a0a_ref[...], a0b_ref[...],
                         a1a_ref[...], a1b_ref[...]], axis=1).astype(BF16)
    yc = jnp.dot(x, wc_ref[...].astype(BF16), preferred_element_type=F32) + bc_ref[...]
    yd = jnp.dot(x, wd_ref[...].astype(BF16), preferred_element_type=F32) + bd_ref[...]
    m = md_ref[...]
    out_ref[...] = m * yc + (1.0 - m) * yd + nf_ref[...]


def _node_mlp(nf, agga, aggb, Wc, Wd, bc, bd, md, N, D, BN):
    nb = N // BN
    h = D // 2
    return pl.pallas_call(
        _node_body,
        grid=(nb,),
        in_specs=[
            pl.BlockSpec((BN, D), lambda i: (i, 0)),
            pl.BlockSpec((BN, h), lambda i: (i, 0)),
            pl.BlockSpec((BN, h), lambda i: (i, 0)),
            pl.BlockSpec((BN, h), lambda i: (i + nb, 0)),
            pl.BlockSpec((BN, h), lambda i: (i + nb, 0)),
            pl.BlockSpec((3 * D, D), lambda i: (0, 0)),
            pl.BlockSpec((3 * D, D), lambda i: (0, 0)),
            pl.BlockSpec((1, D), lambda i: (0, 0)),
            pl.BlockSpec((1, D), lambda i: (0, 0)),
            pl.BlockSpec((BN, 1), lambda i: (i, 0)),
        ],
        out_specs=pl.BlockSpec((BN, D), lambda i: (i, 0)),
        out_shape=jax.ShapeDtypeStruct((N, D), F32),
    )(nf, agga, aggb, agga, aggb, Wc, Wd, bc, bd, md)


def kernel(nf, ef, edge_index, etypes, is_dummy,
           W_outer, b_outer, W_inter, b_inter,
           W_cust, b_cust, W_dummy, b_dummy):
    N, D = nf.shape
    E = ef.shape[0]

    src = edge_index[0]
    dst = edge_index[1]
    iu = etypes * N + src
    iv = etypes * N + dst
    md = (is_dummy == 0).astype(F32).reshape(N, 1)
    mt = jnp.broadcast_to((jnp.arange(2 * N) < N).astype(F32).reshape(-1, 1),
                          (2 * N, 16))

    Wef = jnp.stack([W_outer[:D], W_inter[:D]])
    Wu = jnp.stack([W_outer[D:2 * D], W_inter[D:2 * D]])
    Wv = jnp.stack([W_outer[2 * D:], W_inter[2 * D:]])
    bns = jnp.stack([b_outer, b_inter]).reshape(2, 1, D)

    tua, tub, tva, tvb = _make_tables(nf, Wu, Wv, N, D, BN=N // 10)
    efw0, efw1 = _edge_mlp(ef, Wef, bns, E, D, BE=3200)
    uef, agga, aggb = _sc_edges(tua, tub, tva, tvb, iu, iv,
                                efw0, efw1, mt, ef, N, E, D)
    unf = _node_mlp(nf, agga, aggb, W_cust, W_dummy,
                    b_cust.reshape(1, D), b_dummy.reshape(1, D),
                    md, N, D, BN=N // 5)
    return (unf, uef)


# speedup vs baseline: 1.0850x; 1.0061x over previous
import functools

import jax
import jax.numpy as jnp
from jax import lax
from jax.experimental import pallas as pl
from jax.experimental.pallas import tpu as pltpu
from jax.experimental.pallas import tpu_sc as plsc

F32 = jnp.float32
BF16 = jnp.bfloat16


def _tables_body(nf_ref, wu_ref, wv_ref, tua_ref, tub_ref, tva_ref, tvb_ref):
    x = nf_ref[...].astype(BF16)
    yu = jnp.dot(x, wu_ref[0].astype(BF16), preferred_element_type=F32)
    yv = jnp.dot(x, wv_ref[0].astype(BF16), preferred_element_type=F32)
    h = yu.shape[1] // 2
    tua_ref[...] = yu[:, :h]
    tub_ref[...] = yu[:, h:]
    tva_ref[...] = yv[:, :h]
    tvb_ref[...] = yv[:, h:]


def _make_tables(nf, Wu, Wv, N, D, BN):
    nb = N // BN
    out = jax.ShapeDtypeStruct((2 * N, D // 2), F32)
    return pl.pallas_call(
        _tables_body,
        grid=(2, nb),
        in_specs=[
            pl.BlockSpec((BN, D), lambda t, i: (i, 0)),
            pl.BlockSpec((1, D, D), lambda t, i: (t, 0, 0)),
            pl.BlockSpec((1, D, D), lambda t, i: (t, 0, 0)),
        ],
        out_specs=[
            pl.BlockSpec((BN, D // 2), lambda t, i: (t * nb + i, 0)),
            pl.BlockSpec((BN, D // 2), lambda t, i: (t * nb + i, 0)),
            pl.BlockSpec((BN, D // 2), lambda t, i: (t * nb + i, 0)),
            pl.BlockSpec((BN, D // 2), lambda t, i: (t * nb + i, 0)),
        ],
        out_shape=[out, out, out, out],
    )(nf, Wu, Wv)


def _edge_body(ef_ref, wef_ref, b_ref, out0_ref, out1_ref):
    x = ef_ref[...].astype(BF16)
    out0_ref[...] = jnp.dot(x, wef_ref[0].astype(BF16),
                            preferred_element_type=F32) + b_ref[0]
    out1_ref[...] = jnp.dot(x, wef_ref[1].astype(BF16),
                            preferred_element_type=F32) + b_ref[1]


def _edge_mlp(ef, Wef, bns, E, D, BE):
    out = jax.ShapeDtypeStruct((E, D), F32)
    return pl.pallas_call(
        _edge_body,
        grid=(E // BE,),
        in_specs=[
            pl.BlockSpec((BE, D), lambda i: (i, 0)),
            pl.BlockSpec((2, D, D), lambda i: (0, 0, 0)),
            pl.BlockSpec((2, 1, D), lambda i: (0, 0, 0)),
        ],
        out_specs=[
            pl.BlockSpec((BE, D), lambda i: (i, 0)),
            pl.BlockSpec((BE, D), lambda i: (i, 0)),
        ],
        out_shape=[out, out],
    )(ef, Wef, bns)


def _sc_edge_body(N, E, D, C, ZR, R,
                  tua, tub, tva, tvb, iu_h, iv_h, efw0_h, efw1_h, mt_h,
                  ef_h, zer_h,
                  uef_h, agga_h, aggb_h,
                  iu_t, iv_t, tu_v, tv_v, ew_v, ew1_v, mk_v, ef_v,
                  msg_v, uef_v, agg_sh,
                  sem_gi0, sem_gi1, sem_ge0, sem_ge1,
                  sem_om0, sem_om1, sem_os0, sem_os1, sem_ix):
    H = D // 2
    cid = lax.axis_index("c")
    sid = lax.axis_index("s")
    ept = E // 16
    nchunk = ept // C
    nz = (2 * N) // ZR
    nzq = (nz + 15) // 16
    sem_gi = (sem_gi0, sem_gi1)
    sem_ge = (sem_ge0, sem_ge1)
    sem_om = (sem_om0, sem_om1)
    sem_os = (sem_os0, sem_os1)

    def zc(q, _):
        qq = sid + q * 16

        @pl.when(qq < nz)
        def _():
            pltpu.sync_copy(zer_h, agg_sh.at[pl.ds(qq * ZR, ZR)])
        return 0
    lax.fori_loop(0, nzq, zc, 0)
    plsc.subcore_barrier()

    def pipeline(tab_u, tab_v, coff):
        row0 = sid * nchunk

        def iblock(b):
            return row0 + b * R, b % 3

        pltpu.sync_copy(iu_h.at[pl.ds(row0, R)], iu_t.at[0])
        pltpu.sync_copy(iv_h.at[pl.ds(row0, R)], iv_t.at[0])
        r1, s1 = iblock(1)
        pltpu.async_copy(iu_h.at[pl.ds(r1, R)], iu_t.at[s1], sem_ix)
        pltpu.async_copy(iv_h.at[pl.ds(r1, R)], iv_t.at[s1], sem_ix)

        def irow(k):
            return (k // R) % 3, k % R

        def gstart(k, p):
            base = sid * ept + k * C
            s, r = irow(k)
            pltpu.async_copy(tab_u.at[iu_t.at[s, r]], tu_v.at[p], sem_gi[p])
            pltpu.async_copy(tab_v.at[iv_t.at[s, r]], tv_v.at[p], sem_gi[p])
            pltpu.async_copy(mt_h.at[iu_t.at[s, r]], mk_v.at[p], sem_gi[p])
            pltpu.async_copy(efw0_h.at[pl.ds(base, C), pl.ds(coff, H)],
                             ew_v.at[p], sem_ge[p])
            pltpu.async_copy(efw1_h.at[pl.ds(base, C), pl.ds(coff, H)],
                             ew1_v.at[p], sem_ge[p])
            pltpu.async_copy(ef_h.at[pl.ds(base, C), pl.ds(coff, H)],
                             ef_v.at[p], sem_ge[p])

        def gdrain(k, p):
            base = sid * ept + k * C
            s, r = irow(k)
            pltpu.make_async_copy(tab_u.at[iu_t.at[s, r]], tu_v.at[p],
                                  sem_gi[p]).wait()
            pltpu.make_async_copy(tab_v.at[iv_t.at[s, r]], tv_v.at[p],
                                  sem_gi[p]).wait()
            pltpu.make_async_copy(mt_h.at[iu_t.at[s, r]], mk_v.at[p],
                                  sem_gi[p]).wait()
            pltpu.make_async_copy(efw0_h.at[pl.ds(base, C), pl.ds(coff, H)],
                                  ew_v.at[p], sem_ge[p]).wait()
            pltpu.make_async_copy(efw1_h.at[pl.ds(base, C), pl.ds(coff, H)],
                                  ew1_v.at[p], sem_ge[p]).wait()
            pltpu.make_async_copy(ef_h.at[pl.ds(base, C), pl.ds(coff, H)],
                                  ef_v.at[p], sem_ge[p]).wait()

        def ostart(k, p):
            base = sid * ept + k * C
            s, r = irow(k)
            pltpu.async_copy(uef_v.at[p],
                             uef_h.at[pl.ds(base, C), pl.ds(coff, H)],
                             sem_om[p])
            pltpu.async_copy(msg_v.at[p], agg_sh.at[iv_t.at[s, r]],
                             sem_os[p], add=True)

        def odrain(k, p):
            base = sid * ept + k * C
            s, r = irow(k)
            pltpu.make_async_copy(uef_v.at[p],
                                  uef_h.at[pl.ds(base, C), pl.ds(coff, H)],
                                  sem_om[p]).wait()
            pltpu.make_async_copy(msg_v.at[p], agg_sh.at[iv_t.at[s, r]],
                                  sem_os[p]).wait()

        def compute(p):
            def row(i, _):
                mb = mk_v[p, i, :] > 0.5
                for j in range(H // 16):
                    sl = pl.ds(j * 16, 16)
                    ew = jnp.where(mb, ew_v[p, i, sl], ew1_v[p, i, sl])
                    m = ew + tu_v[p, i, sl] + tv_v[p, i, sl]
                    msg_v[p, i, sl] = m
                    uef_v[p, i, sl] = m + ef_v[p, i, sl]
                return 0
            lax.fori_loop(0, C, row, 0)

        gstart(0, 0)

        def step(kk, _):
            k0 = kk * 2
            k1 = k0 + 1

            @pl.when(jnp.logical_and(lax.rem(k0, R) == R - 2,
                                     k0 + 2 < nchunk))
            def _():
                rb, sb = iblock((k0 + 2) // R)
                pltpu.make_async_copy(iu_h.at[pl.ds(rb, R)], iu_t.at[sb],
                                      sem_ix).wait()
                pltpu.make_async_copy(iv_h.at[pl.ds(rb, R)], iv_t.at[sb],
                                      sem_ix).wait()

            @pl.when(jnp.logical_and(lax.rem(k0, R) == 0,
                                     k0 + R < nchunk))
            def _():
                rb, sb = iblock(k0 // R + 1)
                pltpu.async_copy(iu_h.at[pl.ds(rb, R)], iu_t.at[sb], sem_ix)
                pltpu.async_copy(iv_h.at[pl.ds(rb, R)], iv_t.at[sb], sem_ix)

            @pl.when(kk >= 1)
            def _():
                odrain(k0 - 2, 0)
            gstart(k1, 1)
            gdrain(k0, 0)
            compute(0)
            ostart(k0, 0)

            @pl.when(kk >= 1)
            def _():
                odrain(k1 - 2, 1)

            @pl.when(kk < nchunk // 2 - 1)
            def _():
                gstart(k1 + 1, 0)
            gdrain(k1, 1)
            compute(1)
            ostart(k1, 1)
            return 0
        lax.fori_loop(0, nchunk // 2, step, 0)
        odrain(nchunk - 2, 0)
        odrain(nchunk - 1, 1)

    @pl.when(cid == 0)
    def _():
        pipeline(tua, tva, 0)

    @pl.when(cid == 1)
    def _():
        pipeline(tub, tvb, H)

    plsc.subcore_barrier()

    def co(q, _):
        qq = sid + q * 16

        @pl.when(qq < nz)
        def _():
            sl = pl.ds(qq * ZR, ZR)

            @pl.when(cid == 0)
            def _():
                pltpu.sync_copy(agg_sh.at[sl], agga_h.at[sl])

            @pl.when(cid == 1)
            def _():
                pltpu.sync_copy(agg_sh.at[sl], aggb_h.at[sl])
        return 0
    lax.fori_loop(0, nzq, co, 0)


def _sc_edges(tua, tub, tva, tvb, iu, iv, efw0, efw1, mt, ef, N, E, D):
    C = 50
    ZR = 200
    R = 10
    H = D // 2
    iu2 = iu.reshape(E // C, C)
    iv2 = iv.reshape(E // C, C)
    zer = jnp.zeros((ZR, H), F32)
    mesh = plsc.VectorSubcoreMesh(core_axis_name="c", subcore_axis_name="s",
                                  num_cores=2, num_subcores=16)
    eh = jax.ShapeDtypeStruct((E, D), F32)
    ah = jax.ShapeDtypeStruct((2 * N, H), F32)
    f = pl.kernel(
        functools.partial(_sc_edge_body, N, E, D, C, ZR, R),
        out_type=[eh, ah, ah],
        mesh=mesh,
        compiler_params=pltpu.CompilerParams(use_tc_tiling_on_sc=False),
        scratch_types=[
            pltpu.VMEM((3, R, C), jnp.int32),
            pltpu.VMEM((3, R, C), jnp.int32),
            pltpu.VMEM((2, C, H), F32),
            pltpu.VMEM((2, C, H), F32),
            pltpu.VMEM((2, C, H), F32),
            pltpu.VMEM((2, C, H), F32),
            pltpu.VMEM((2, C, 16), F32),
            pltpu.VMEM((2, C, H), F32),
            pltpu.VMEM((2, C, H), F32),
            pltpu.VMEM((2, C, H), F32),
            pltpu.VMEM_SHARED((2 * N, H), F32),
            pltpu.SemaphoreType.DMA,
            pltpu.SemaphoreType.DMA,
            pltpu.SemaphoreType.DMA,
            pltpu.SemaphoreType.DMA,
            pltpu.SemaphoreType.DMA,
            pltpu.SemaphoreType.DMA,
            pltpu.SemaphoreType.DMA,
            pltpu.SemaphoreType.DMA,
            pltpu.SemaphoreType.DMA,
        ],
    )
    return f(tua, tub, tva, tvb, iu2, iv2, efw0, efw1, mt, ef, zer)


def _uef_body(ef_ref, m_ref, out_ref):
    out_ref[...] = m_ref[...] + ef_ref[...]


def _uef_add(ef, msg, E, D, BE):
    return pl.pallas_call(
        _uef_body,
        grid=(E // BE,),
        in_specs=[
            pl.BlockSpec((BE, D), lambda i: (i, 0)),
            pl.BlockSpec((BE, D), lambda i: (i, 0)),
        ],
        out_specs=pl.BlockSpec((BE, D), lambda i: (i, 0)),
        out_shape=jax.ShapeDtypeStruct((E, D), F32),
    )(ef, msg)


def _node_body(nf_ref, a0a_ref, a0b_ref, a1a_ref, a1b_ref,
               wc_ref, wd_ref, bc_ref, bd_ref, md_ref, out_ref):
    x = jnp.concatenate([nf_ref[...], a0a_ref[...], a0b_ref[...],
                         a1a_ref[...], a1b_ref[...]], axis=1).astype(BF16)
    yc = jnp.dot(x, wc_ref[...].astype(BF16), preferred_element_type=F32) + bc_ref[...]
    yd = jnp.dot(x, wd_ref[...].astype(BF16), preferred_element_type=F32) + bd_ref[...]
    m = md_ref[...]
    out_ref[...] = m * yc + (1.0 - m) * yd + nf_ref[...]


def _node_mlp(nf, agga, aggb, Wc, Wd, bc, bd, md, N, D, BN):
    nb = N // BN
    h = D // 2
    return pl.pallas_call(
        _node_body,
        grid=(nb,),
        in_specs=[
            pl.BlockSpec((BN, D), lambda i: (i, 0)),
            pl.BlockSpec((BN, h), lambda i: (i, 0)),
            pl.BlockSpec((BN, h), lambda i: (i, 0)),
            pl.BlockSpec((BN, h), lambda i: (i + nb, 0)),
            pl.BlockSpec((BN, h), lambda i: (i + nb, 0)),
            pl.BlockSpec((3 * D, D), lambda i: (0, 0)),
            pl.BlockSpec((3 * D, D), lambda i: (0, 0)),
            pl.BlockSpec((1, D), lambda i: (0, 0)),
            pl.BlockSpec((1, D), lambda i: (0, 0)),
            pl.BlockSpec((BN, 1), lambda i: (i, 0)),
        ],
        out_specs=pl.BlockSpec((BN, D), lambda i: (i, 0)),
        out_shape=jax.ShapeDtypeStruct((N, D), F32),
    )(nf, agga, aggb, agga, aggb, Wc, Wd, bc, bd, md)


def kernel(nf, ef, edge_index, etypes, is_dummy,
           W_outer, b_outer, W_inter, b_inter,
           W_cust, b_cust, W_dummy, b_dummy):
    N, D = nf.shape
    E = ef.shape[0]

    src = edge_index[0]
    dst = edge_index[1]
    iu = etypes * N + src
    iv = etypes * N + dst
    md = (is_dummy == 0).astype(F32).reshape(N, 1)
    mt = jnp.broadcast_to((jnp.arange(2 * N) < N).astype(F32).reshape(-1, 1),
                          (2 * N, 16))

    Wef = jnp.stack([W_outer[:D], W_inter[:D]])
    Wu = jnp.stack([W_outer[D:2 * D], W_inter[D:2 * D]])
    Wv = jnp.stack([W_outer[2 * D:], W_inter[2 * D:]])
    bns = jnp.stack([b_outer, b_inter]).reshape(2, 1, D)

    tua, tub, tva, tvb = _make_tables(nf, Wu, Wv, N, D, BN=N // 10)
    efw0, efw1 = _edge_mlp(ef, Wef, bns, E, D, BE=3200)
    uef, agga, aggb = _sc_edges(tua, tub, tva, tvb, iu, iv,
                                efw0, efw1, mt, ef, N, E, D)
    unf = _node_mlp(nf, agga, aggb, W_cust, W_dummy,
                    b_cust.reshape(1, D), b_dummy.reshape(1, D),
                    md, N, D, BN=N // 5)
    return (unf, uef)
